# baseline (speedup 1.0000x reference)
import functools

import jax
import jax.numpy as jnp
from jax import lax
from jax.experimental import pallas as pl
from jax.experimental.pallas import tpu as pltpu
from jax.experimental.pallas import tpu_sc as plsc

N = 10000
E = 320000
F = 128

_NC = 2
_NS = 16
_L = 16
_NW = _NC * _NS
_EPT = (E // (_NW * 128)) * 128
_REM = E - _NW * _EPT
_RBASE = _NW * _EPT
_NCHUNK = 6
_CSZ = _EPT // _NCHUNK
_UNROLL = 8
_GPC = _CSZ // (_UNROLL * _L)


def _proj_body(att_ref, ns_ref, sp_ref, tp_ref):
    r = lax.dot_general(
        att_ref[...],
        ns_ref[...],
        dimension_numbers=(((1,), (1,)), ((), ())),
        preferred_element_type=jnp.float32,
    )
    sp_ref[...] = r[0]
    tp_ref[...] = r[1]


def _project(att2, node_signal):
    return pl.pallas_call(
        _proj_body,
        out_shape=[
            jax.ShapeDtypeStruct((N,), jnp.float32),
            jax.ShapeDtypeStruct((N,), jnp.float32),
        ],
    )(att2, node_signal)


@functools.partial(
    pl.kernel,
    out_type=jax.ShapeDtypeStruct((E,), jnp.float32),
    mesh=plsc.VectorSubcoreMesh(core_axis_name="c", subcore_axis_name="s"),
    compiler_params=pltpu.CompilerParams(needs_layout_passes=False),
    scratch_types=[
        pltpu.VMEM((N,), jnp.float32),
        pltpu.VMEM((N,), jnp.float32),
        pltpu.VMEM((2, _EPT + _REM), jnp.int32),
        pltpu.VMEM((_EPT + _REM,), jnp.float32),
        pltpu.VMEM_SHARED((N,), jnp.float32),
        pltpu.VMEM_SHARED((N,), jnp.float32),
        pltpu.SemaphoreType.DMA,
        [pltpu.SemaphoreType.DMA] * _NCHUNK,
        pltpu.SemaphoreType.DMA,
        pltpu.SemaphoreType.DMA,
    ],
)
def _edge_sc(sp_hbm, tp_hbm, ei_hbm, out_hbm,
             sp_v, tp_v, ev_v, out_v, sh_sp, sh_tp,
             sem_t, sem_ev, sem_out, sem_rem):
    wid = lax.axis_index("s") * _NC + lax.axis_index("c")
    base = pl.multiple_of(wid * _EPT, 128)
    last = wid == _NW - 1

    sid = lax.axis_index("s")
    ev_copies = []
    for j in range(_NCHUNK):
        ev_copies.append(pltpu.async_copy(
            ei_hbm.at[:, pl.ds(base + j * _CSZ, _CSZ)],
            ev_v.at[:, pl.ds(j * _CSZ, _CSZ)], sem_ev[j]))

    @pl.when(last)
    def _start_rem():
        pltpu.async_copy(ei_hbm.at[:, pl.ds(_RBASE, _REM)],
                         ev_v.at[:, pl.ds(_EPT, _REM)], sem_rem)

    @pl.when(sid == 0)
    def _stage_tables():
        pltpu.async_copy(sp_hbm, sh_sp, sem_t)
        pltpu.async_copy(tp_hbm, sh_tp, sem_t)
        pltpu.make_async_copy(sp_hbm, sh_sp, sem_t).wait()
        pltpu.make_async_copy(tp_hbm, sh_tp, sem_t).wait()

    plsc.subcore_barrier()
    pltpu.async_copy(sh_sp, sp_v, sem_t)
    pltpu.async_copy(sh_tp, tp_v, sem_t)
    pltpu.make_async_copy(sh_sp, sp_v, sem_t).wait()
    pltpu.make_async_copy(sh_tp, tp_v, sem_t).wait()

    def gather16(off):
        s = plsc.load_gather(sp_v, [ev_v[0, pl.ds(off, _L)]])
        t = plsc.load_gather(tp_v, [ev_v[1, pl.ds(off, _L)]])
        out_v[pl.ds(off, _L)] = jnp.maximum(s + t, 0.0)

    for j in range(_NCHUNK):
        ev_copies[j].wait()

        @plsc.parallel_loop(j * _CSZ, (j + 1) * _CSZ, _L, unroll=_UNROLL)
        def _chunk(i):
            gather16(pl.multiple_of(i, _L))

        pltpu.async_copy(out_v.at[pl.ds(j * _CSZ, _CSZ)],
                         out_hbm.at[pl.ds(base + j * _CSZ, _CSZ)], sem_out)

    @pl.when(last)
    def _finish_rem():
        pltpu.make_async_copy(ei_hbm.at[:, pl.ds(_RBASE, _REM)],
                              ev_v.at[:, pl.ds(_EPT, _REM)], sem_rem).wait()

        @plsc.parallel_loop(_EPT, _EPT + _REM, _L, unroll=_UNROLL)
        def _rem(i):
            gather16(pl.multiple_of(i, _L))

        pltpu.async_copy(out_v.at[pl.ds(_EPT, _REM)],
                         out_hbm.at[pl.ds(_RBASE, _REM)], sem_out)
        pltpu.make_async_copy(out_v.at[pl.ds(_EPT, _REM)],
                              out_hbm.at[pl.ds(_RBASE, _REM)], sem_out).wait()

    for j in range(_NCHUNK):
        pltpu.make_async_copy(out_v.at[pl.ds(j * _CSZ, _CSZ)],
                              out_hbm.at[pl.ds(base + j * _CSZ, _CSZ)],
                              sem_out).wait()


@jax.jit
def kernel(node_signal, edge_index, att):
    att2 = att.reshape(2, F)
    sp, tp = _project(att2, node_signal)
    return _edge_sc(sp, tp, edge_index).reshape(E, 1)


# speedup vs baseline: 1.0066x; 1.0066x over previous
import functools

import jax
import jax.numpy as jnp
from jax import lax
from jax.experimental import pallas as pl
from jax.experimental.pallas import tpu as pltpu
from jax.experimental.pallas import tpu_sc as plsc

N = 10000
E = 320000
F = 128

_NC = 2
_NS = 16
_L = 16
_NW = _NC * _NS
_EPT = (E // (_NW * 128)) * 128
_REM = E - _NW * _EPT
_RBASE = _NW * _EPT
_NCHUNK = 3
_CSZ = _EPT // _NCHUNK
_UNROLL = 4
_GPC = _CSZ // (_UNROLL * _L)


def _proj_body(att_ref, ns_ref, sp_ref, tp_ref):
    r = lax.dot_general(
        att_ref[...],
        ns_ref[...],
        dimension_numbers=(((1,), (1,)), ((), ())),
        preferred_element_type=jnp.float32,
    )
    sp_ref[...] = r[0]
    tp_ref[...] = r[1]


def _project(att2, node_signal):
    return pl.pallas_call(
        _proj_body,
        out_shape=[
            jax.ShapeDtypeStruct((N,), jnp.float32),
            jax.ShapeDtypeStruct((N,), jnp.float32),
        ],
    )(att2, node_signal)


@functools.partial(
    pl.kernel,
    out_type=jax.ShapeDtypeStruct((E,), jnp.float32),
    mesh=plsc.VectorSubcoreMesh(core_axis_name="c", subcore_axis_name="s"),
    compiler_params=pltpu.CompilerParams(needs_layout_passes=False),
    scratch_types=[
        pltpu.VMEM((N,), jnp.float32),
        pltpu.VMEM((N,), jnp.float32),
        pltpu.VMEM((2, _EPT + _REM), jnp.int32),
        pltpu.VMEM((_EPT + _REM,), jnp.float32),
        pltpu.VMEM_SHARED((N,), jnp.float32),
        pltpu.VMEM_SHARED((N,), jnp.float32),
        pltpu.SemaphoreType.DMA,
        [pltpu.SemaphoreType.DMA] * _NCHUNK,
        pltpu.SemaphoreType.DMA,
        pltpu.SemaphoreType.DMA,
    ],
)
def _edge_sc(sp_hbm, tp_hbm, ei_hbm, out_hbm,
             sp_v, tp_v, ev_v, out_v, sh_sp, sh_tp,
             sem_t, sem_ev, sem_out, sem_rem):
    wid = lax.axis_index("s") * _NC + lax.axis_index("c")
    base = pl.multiple_of(wid * _EPT, 128)
    last = wid == _NW - 1

    sid = lax.axis_index("s")
    ev_copies = []
    for j in range(_NCHUNK):
        ev_copies.append(pltpu.async_copy(
            ei_hbm.at[:, pl.ds(base + j * _CSZ, _CSZ)],
            ev_v.at[:, pl.ds(j * _CSZ, _CSZ)], sem_ev[j]))

    @pl.when(last)
    def _start_rem():
        pltpu.async_copy(ei_hbm.at[:, pl.ds(_RBASE, _REM)],
                         ev_v.at[:, pl.ds(_EPT, _REM)], sem_rem)

    @pl.when(sid == 0)
    def _stage_tables():
        pltpu.async_copy(sp_hbm, sh_sp, sem_t)
        pltpu.async_copy(tp_hbm, sh_tp, sem_t)
        pltpu.make_async_copy(sp_hbm, sh_sp, sem_t).wait()
        pltpu.make_async_copy(tp_hbm, sh_tp, sem_t).wait()

    plsc.subcore_barrier()
    pltpu.async_copy(sh_sp, sp_v, sem_t)
    pltpu.async_copy(sh_tp, tp_v, sem_t)
    pltpu.make_async_copy(sh_sp, sp_v, sem_t).wait()
    pltpu.make_async_copy(sh_tp, tp_v, sem_t).wait()

    def gather16(off):
        s = plsc.load_gather(sp_v, [ev_v[0, pl.ds(off, _L)]])
        t = plsc.load_gather(tp_v, [ev_v[1, pl.ds(off, _L)]])
        out_v[pl.ds(off, _L)] = jnp.maximum(s + t, 0.0)

    for j in range(_NCHUNK):
        ev_copies[j].wait()

        @plsc.parallel_loop(j * _CSZ, (j + 1) * _CSZ, _L, unroll=_UNROLL)
        def _chunk(i):
            gather16(pl.multiple_of(i, _L))

        pltpu.async_copy(out_v.at[pl.ds(j * _CSZ, _CSZ)],
                         out_hbm.at[pl.ds(base + j * _CSZ, _CSZ)], sem_out)

    @pl.when(last)
    def _finish_rem():
        pltpu.make_async_copy(ei_hbm.at[:, pl.ds(_RBASE, _REM)],
                              ev_v.at[:, pl.ds(_EPT, _REM)], sem_rem).wait()

        @plsc.parallel_loop(_EPT, _EPT + _REM, _L, unroll=_UNROLL)
        def _rem(i):
            gather16(pl.multiple_of(i, _L))

        pltpu.async_copy(out_v.at[pl.ds(_EPT, _REM)],
                         out_hbm.at[pl.ds(_RBASE, _REM)], sem_out)
        pltpu.make_async_copy(out_v.at[pl.ds(_EPT, _REM)],
                              out_hbm.at[pl.ds(_RBASE, _REM)], sem_out).wait()

    for j in range(_NCHUNK):
        pltpu.make_async_copy(out_v.at[pl.ds(j * _CSZ, _CSZ)],
                              out_hbm.at[pl.ds(base + j * _CSZ, _CSZ)],
                              sem_out).wait()


@jax.jit
def kernel(node_signal, edge_index, att):
    att2 = att.reshape(2, F)
    sp, tp = _project(att2, node_signal)
    return _edge_sc(sp, tp, edge_index).reshape(E, 1)


# speedup vs baseline: 1.0097x; 1.0030x over previous
import functools

import jax
import jax.numpy as jnp
from jax import lax
from jax.experimental import pallas as pl
from jax.experimental.pallas import tpu as pltpu
from jax.experimental.pallas import tpu_sc as plsc

N = 10000
E = 320000
F = 128

_NC = 2
_NS = 16
_L = 16
_NW = _NC * _NS
_EPT = (E // (_NW * 128)) * 128
_REM = E - _NW * _EPT
_RBASE = _NW * _EPT
_NCHUNK = 3
_CSZ = _EPT // _NCHUNK
_UNROLL = 4


def _proj_body(att_ref, ns_ref, sp_ref, tp_ref):
    r = lax.dot_general(
        att_ref[...],
        ns_ref[...],
        dimension_numbers=(((1,), (1,)), ((), ())),
        preferred_element_type=jnp.float32,
    )
    sp_ref[...] = r[0]
    tp_ref[...] = r[1]


def _project(att2, node_signal):
    return pl.pallas_call(
        _proj_body,
        out_shape=[
            jax.ShapeDtypeStruct((N,), jnp.float32),
            jax.ShapeDtypeStruct((N,), jnp.float32),
        ],
    )(att2, node_signal)


@functools.partial(
    pl.kernel,
    out_type=jax.ShapeDtypeStruct((E,), jnp.float32),
    mesh=plsc.VectorSubcoreMesh(core_axis_name="c", subcore_axis_name="s"),
    compiler_params=pltpu.CompilerParams(needs_layout_passes=False),
    scratch_types=[
        pltpu.VMEM((N,), jnp.float32),
        pltpu.VMEM((N,), jnp.float32),
        pltpu.VMEM((2, _EPT + _REM), jnp.int32),
        pltpu.VMEM((_EPT + _REM,), jnp.float32),
        pltpu.VMEM_SHARED((N,), jnp.float32),
        pltpu.VMEM_SHARED((N,), jnp.float32),
        pltpu.SemaphoreType.DMA,
        [pltpu.SemaphoreType.DMA] * _NCHUNK,
        pltpu.SemaphoreType.DMA,
        pltpu.SemaphoreType.DMA,
    ],
)
def _edge_sc(sp_hbm, tp_hbm, ei_hbm, out_hbm,
             sp_v, tp_v, ev_v, out_v, sh_sp, sh_tp,
             sem_t, sem_ev, sem_out, sem_rem):
    wid = lax.axis_index("s") * _NC + lax.axis_index("c")
    base = pl.multiple_of(wid * _EPT, 128)
    last = wid == _NW - 1

    sid = lax.axis_index("s")
    ev_copies = []
    for j in range(_NCHUNK):
        ev_copies.append(pltpu.async_copy(
            ei_hbm.at[:, pl.ds(base + j * _CSZ, _CSZ)],
            ev_v.at[:, pl.ds(j * _CSZ, _CSZ)], sem_ev[j]))

    @pl.when(last)
    def _start_rem():
        pltpu.async_copy(ei_hbm.at[:, pl.ds(_RBASE, _REM)],
                         ev_v.at[:, pl.ds(_EPT, _REM)], sem_rem)

    @pl.when(sid == 0)
    def _stage_tables():
        pltpu.async_copy(sp_hbm, sh_sp, sem_t)
        pltpu.async_copy(tp_hbm, sh_tp, sem_t)
        pltpu.make_async_copy(sp_hbm, sh_sp, sem_t).wait()
        pltpu.make_async_copy(tp_hbm, sh_tp, sem_t).wait()

    plsc.subcore_barrier()
    pltpu.async_copy(sh_sp, sp_v, sem_t)
    pltpu.async_copy(sh_tp, tp_v, sem_t)
    pltpu.make_async_copy(sh_sp, sp_v, sem_t).wait()
    pltpu.make_async_copy(sh_tp, tp_v, sem_t).wait()

    def gather16(off):
        s = plsc.load_gather(sp_v, [ev_v[0, pl.ds(off, _L)]])
        t = plsc.load_gather(tp_v, [ev_v[1, pl.ds(off, _L)]])
        out_v[pl.ds(off, _L)] = jnp.maximum(s + t, 0.0)

    for j in range(_NCHUNK):
        ev_copies[j].wait()

        @plsc.parallel_loop(j * _CSZ, (j + 1) * _CSZ, _L, unroll=_UNROLL)
        def _chunk(i):
            gather16(pl.multiple_of(i, _L))

        pltpu.async_copy(out_v.at[pl.ds(j * _CSZ, _CSZ)],
                         out_hbm.at[pl.ds(base + j * _CSZ, _CSZ)], sem_out)

    @pl.when(last)
    def _finish_rem():
        pltpu.make_async_copy(ei_hbm.at[:, pl.ds(_RBASE, _REM)],
                              ev_v.at[:, pl.ds(_EPT, _REM)], sem_rem).wait()

        @plsc.parallel_loop(_EPT, _EPT + _REM, _L, unroll=_UNROLL)
        def _rem(i):
            gather16(pl.multiple_of(i, _L))

        pltpu.async_copy(out_v.at[pl.ds(_EPT, _REM)],
                         out_hbm.at[pl.ds(_RBASE, _REM)], sem_out)
        pltpu.make_async_copy(out_v.at[pl.ds(_EPT, _REM)],
                              out_hbm.at[pl.ds(_RBASE, _REM)], sem_out).wait()

    for j in range(_NCHUNK):
        pltpu.make_async_copy(out_v.at[pl.ds(j * _CSZ, _CSZ)],
                              out_hbm.at[pl.ds(base + j * _CSZ, _CSZ)],
                              sem_out).wait()


@jax.jit
def kernel(node_signal, edge_index, att):
    att2 = att.reshape(2, F)
    sp, tp = _project(att2, node_signal)
    return _edge_sc(sp, tp, edge_index).reshape(E, 1)
